# Initial kernel scaffold; baseline (speedup 1.0000x reference)
import jax
import jax.numpy as jnp
from jax.experimental import pallas as pl


def kernel(x, edge_index, W1, att_src1, att_dst1, b1, W2, att_src2, att_dst2, b2):
    raise NotImplementedError("write your pallas kernel here")


# speedup vs baseline: 23.3089x; 23.3089x over previous
import functools

import jax
import jax.numpy as jnp
from jax import lax
from jax.experimental import pallas as pl
from jax.experimental.pallas import tpu as pltpu
from jax.experimental.pallas import tpu_sc as plsc

N = 50000
IN_DIM = 1433
NC, NS, L = 2, 16, 16
NT = NC * NS
RB = 512
N_PAD = 50176
E = N + 800000
E_PAD = 851968
EQ = E_PAD // 4
EH = E_PAD // 2
CH = 512
CHM = 1024
_CP = pltpu.CompilerParams(needs_layout_passes=False)
_MESH = plsc.VectorSubcoreMesh(core_axis_name="c", subcore_axis_name="s",
                               num_cores=NC, num_subcores=NS)


def _tc1_body(x_ref, w1_ref, s1_ref, d1_ref, ht_ref, ast_ref, adt_ref):
    h = jnp.dot(x_ref[...], w1_ref[...], preferred_element_type=jnp.float32)
    asrc = jnp.dot(h, s1_ref[...], preferred_element_type=jnp.float32)
    adst = jnp.dot(h, d1_ref[...], preferred_element_type=jnp.float32)
    ht_ref[...] = jnp.transpose(h)
    ast_ref[...] = jnp.transpose(asrc)
    adt_ref[...] = jnp.transpose(adst)


_tc1 = pl.pallas_call(
    _tc1_body,
    grid=(N_PAD // RB,),
    in_specs=[
        pl.BlockSpec((RB, IN_DIM), lambda i: (i, 0)),
        pl.BlockSpec((IN_DIM, 64), lambda i: (0, 0)),
        pl.BlockSpec((64, 8), lambda i: (0, 0)),
        pl.BlockSpec((64, 8), lambda i: (0, 0)),
    ],
    out_specs=[
        pl.BlockSpec((64, RB), lambda i: (0, i)),
        pl.BlockSpec((8, RB), lambda i: (0, i)),
        pl.BlockSpec((8, RB), lambda i: (0, i)),
    ],
    out_shape=[
        jax.ShapeDtypeStruct((64, N_PAD), jnp.float32),
        jax.ShapeDtypeStruct((8, N_PAD), jnp.float32),
        jax.ShapeDtypeStruct((8, N_PAD), jnp.float32),
    ],
)


def _tc2_body(acc_ref, sp_ref, b1_ref, rep_ref, w2_ref, s2_ref, d2_ref,
              h2t_ref, a2t_ref):
    a = acc_ref[...]
    num = jnp.transpose(a[0] + a[1])
    sp = sp_ref[...]
    sden = jnp.transpose(sp[0] + sp[1] + sp[2] + sp[3])
    srep = jnp.dot(sden, rep_ref[...], preferred_element_type=jnp.float32)
    out1 = num / (srep + 1e-16) + b1_ref[...]
    hh = jnp.where(out1 > 0, out1, jnp.exp(out1) - 1.0)
    h2 = jnp.dot(hh, w2_ref[...], preferred_element_type=jnp.float32)
    asrc2 = jnp.dot(h2, s2_ref[...], preferred_element_type=jnp.float32)
    adst2 = jnp.dot(h2, d2_ref[...], preferred_element_type=jnp.float32)
    h2t_ref[...] = jnp.transpose(h2)
    a2t_ref[...] = jnp.transpose(jnp.concatenate([asrc2, adst2], axis=1))


_tc2 = pl.pallas_call(
    _tc2_body,
    grid=(N_PAD // RB,),
    in_specs=[
        pl.BlockSpec((2, 64, RB), lambda i: (0, 0, i)),
        pl.BlockSpec((4, 8, RB), lambda i: (0, 0, i)),
        pl.BlockSpec((1, 64), lambda i: (0, 0)),
        pl.BlockSpec((8, 64), lambda i: (0, 0)),
        pl.BlockSpec((64, 7), lambda i: (0, 0)),
        pl.BlockSpec((7, 1), lambda i: (0, 0)),
        pl.BlockSpec((7, 1), lambda i: (0, 0)),
    ],
    out_specs=[
        pl.BlockSpec((7, RB), lambda i: (0, i)),
        pl.BlockSpec((2, RB), lambda i: (0, i)),
    ],
    out_shape=[
        jax.ShapeDtypeStruct((7, N_PAD), jnp.float32),
        jax.ShapeDtypeStruct((2, N_PAD), jnp.float32),
    ],
)


def _tc3_body(acc_ref, b2_ref, out_ref):
    a = acc_ref[...]
    t = a[0] + a[1] + a[2] + a[3]
    o = jnp.transpose(t[:7] / (t[7:8] + 1e-16)) + b2_ref[...]
    m = jnp.max(o, axis=1, keepdims=True)
    z = o - m
    lse = jnp.log(jnp.sum(jnp.exp(z), axis=1, keepdims=True))
    out_ref[...] = z - lse


_tc3 = pl.pallas_call(
    _tc3_body,
    grid=(N_PAD // RB,),
    in_specs=[
        pl.BlockSpec((4, 8, RB), lambda i: (0, 0, i)),
        pl.BlockSpec((1, 7), lambda i: (0, 0)),
    ],
    out_specs=[pl.BlockSpec((RB, 7), lambda i: (i, 0))],
    out_shape=[jax.ShapeDtypeStruct((N, 7), jnp.float32)],
)


def _zero_1d(ref, nwords):
    z16 = jnp.zeros((L,), jnp.float32)

    def zb(r, _):
        ref[pl.ds(r * L, L)] = z16
        return 0

    lax.fori_loop(0, nwords // L, zb, 0)


@functools.partial(
    pl.kernel,
    out_type=jax.ShapeDtypeStruct((8, E_PAD), jnp.float32),
    mesh=_MESH,
    compiler_params=_CP,
    scratch_types=[
        pltpu.VMEM((N_PAD,), jnp.float32),
        pltpu.VMEM((N_PAD,), jnp.float32),
        pltpu.VMEM((CH,), jnp.int32),
        pltpu.VMEM((CH,), jnp.int32),
        pltpu.VMEM((CH,), jnp.float32),
    ],
)
def _scw1(src_hbm, dst_hbm, ast_hbm, adt_hbm, wt_hbm,
          av, dv, src_v, dst_v, wv):
    wid = lax.axis_index("s") * NC + lax.axis_index("c")
    task_h = wid // 4
    task_q = wid - task_h * 4
    pltpu.sync_copy(ast_hbm.at[task_h], av)
    pltpu.sync_copy(adt_hbm.at[task_h], dv)
    ebase = task_q * EQ

    def chunk(k, _):
        e0 = ebase + k * CH
        pltpu.sync_copy(src_hbm.at[pl.ds(e0, CH)], src_v)
        pltpu.sync_copy(dst_hbm.at[pl.ds(e0, CH)], dst_v)

        def grp(g, _):
            s16 = src_v[pl.ds(g * L, L)]
            d16 = dst_v[pl.ds(g * L, L)]
            xv = plsc.load_gather(av, [s16]) + plsc.load_gather(dv, [d16])
            wv[pl.ds(g * L, L)] = jnp.exp(jnp.maximum(xv, 0.2 * xv))
            return 0

        lax.fori_loop(0, CH // L, grp, 0)
        pltpu.sync_copy(wv, wt_hbm.at[task_h, pl.ds(e0, CH)])
        return 0

    lax.fori_loop(0, EQ // CH, chunk, 0)


@functools.partial(
    pl.kernel,
    out_type=jax.ShapeDtypeStruct((4, 8, N_PAD), jnp.float32),
    mesh=_MESH,
    compiler_params=_CP,
    scratch_types=[
        pltpu.VMEM((N_PAD,), jnp.float32),
        pltpu.VMEM((CH,), jnp.int32),
        pltpu.VMEM((CH,), jnp.float32),
    ],
)
def _scs1(dst_hbm, wt_hbm, out_hbm, accv, dst_v, wv):
    wid = lax.axis_index("s") * NC + lax.axis_index("c")
    task_h = wid // 4
    task_q = wid - task_h * 4
    _zero_1d(accv, N_PAD)
    ebase = task_q * EQ

    def chunk(k, _):
        e0 = ebase + k * CH
        pltpu.sync_copy(dst_hbm.at[pl.ds(e0, CH)], dst_v)
        pltpu.sync_copy(wt_hbm.at[task_h, pl.ds(e0, CH)], wv)

        def grp(g, _):
            d16 = dst_v[pl.ds(g * L, L)]
            w = wv[pl.ds(g * L, L)]
            plsc.addupdate_scatter(accv, [d16], w)
            return 0

        lax.fori_loop(0, CH // L, grp, 0)
        return 0

    lax.fori_loop(0, EQ // CH, chunk, 0)
    pltpu.sync_copy(accv, out_hbm.at[task_q, task_h])


@functools.partial(
    pl.kernel,
    out_type=jax.ShapeDtypeStruct((2, 64, N_PAD), jnp.float32),
    mesh=_MESH,
    compiler_params=_CP,
    scratch_types=[
        pltpu.VMEM((N_PAD,), jnp.float32),
        pltpu.VMEM((N_PAD,), jnp.float32),
        pltpu.VMEM((CHM,), jnp.int32),
        pltpu.VMEM((CHM,), jnp.int32),
        pltpu.VMEM((CHM,), jnp.float32),
    ],
)
def _scm1(src_hbm, dst_hbm, ht_hbm, wt_hbm, out_hbm,
          tabv, accv, src_v, dst_v, wv):
    wid = lax.axis_index("s") * NC + lax.axis_index("c")

    for t in range(4):
        tid = wid * 4 + t
        ch = tid // 2
        half = tid - ch * 2
        head = ch // 8
        _zero_1d(accv, N_PAD)
        pltpu.sync_copy(ht_hbm.at[ch], tabv)
        ebase = half * EH

        def chunk(k, _):
            e0 = ebase + k * CHM
            pltpu.sync_copy(src_hbm.at[pl.ds(e0, CHM)], src_v)
            pltpu.sync_copy(dst_hbm.at[pl.ds(e0, CHM)], dst_v)
            pltpu.sync_copy(wt_hbm.at[head, pl.ds(e0, CHM)], wv)

            def grp(g, _):
                s16 = src_v[pl.ds(g * L, L)]
                d16 = dst_v[pl.ds(g * L, L)]
                w = wv[pl.ds(g * L, L)]
                v = plsc.load_gather(tabv, [s16])
                plsc.addupdate_scatter(accv, [d16], v * w)
                return 0

            lax.fori_loop(0, CHM // L, grp, 0)
            return 0

        lax.fori_loop(0, EH // CHM, chunk, 0)
        pltpu.sync_copy(accv, out_hbm.at[half, ch])


@functools.partial(
    pl.kernel,
    out_type=jax.ShapeDtypeStruct((4, 8, N_PAD), jnp.float32),
    mesh=_MESH,
    compiler_params=_CP,
    scratch_types=[
        pltpu.VMEM((N_PAD,), jnp.float32),
        pltpu.VMEM((N_PAD,), jnp.float32),
        pltpu.VMEM((CH,), jnp.int32),
        pltpu.VMEM((CH,), jnp.int32),
        pltpu.VMEM((CH,), jnp.float32),
    ],
)
def _scm2(src_hbm, dst_hbm, h2t_hbm, w2_hbm, out_hbm,
          tabv, accv, src_v, dst_v, wv):
    wid = lax.axis_index("s") * NC + lax.axis_index("c")
    task_c = wid // 4
    task_q = wid - task_c * 4
    _zero_1d(accv, N_PAD)

    @pl.when(task_c < 7)
    def _():
        pltpu.sync_copy(h2t_hbm.at[task_c], tabv)

    ebase = task_q * EQ

    def chunk(k, _):
        e0 = ebase + k * CH
        pltpu.sync_copy(src_hbm.at[pl.ds(e0, CH)], src_v)
        pltpu.sync_copy(dst_hbm.at[pl.ds(e0, CH)], dst_v)
        pltpu.sync_copy(w2_hbm.at[pl.ds(e0, CH)], wv)

        @pl.when(task_c < 7)
        def _():
            def grp(g, _):
                s16 = src_v[pl.ds(g * L, L)]
                d16 = dst_v[pl.ds(g * L, L)]
                w = wv[pl.ds(g * L, L)]
                v = plsc.load_gather(tabv, [s16])
                plsc.addupdate_scatter(accv, [d16], v * w)
                return 0

            lax.fori_loop(0, CH // L, grp, 0)

        @pl.when(task_c == 7)
        def _():
            def grp(g, _):
                d16 = dst_v[pl.ds(g * L, L)]
                w = wv[pl.ds(g * L, L)]
                plsc.addupdate_scatter(accv, [d16], w)
                return 0

            lax.fori_loop(0, CH // L, grp, 0)

        return 0

    lax.fori_loop(0, EQ // CH, chunk, 0)
    pltpu.sync_copy(accv, out_hbm.at[task_q, task_c])


@functools.partial(
    pl.kernel,
    out_type=jax.ShapeDtypeStruct((E_PAD,), jnp.float32),
    mesh=_MESH,
    compiler_params=_CP,
    scratch_types=[
        pltpu.VMEM((N_PAD,), jnp.float32),
        pltpu.VMEM((N_PAD,), jnp.float32),
        pltpu.VMEM((CH,), jnp.int32),
        pltpu.VMEM((CH,), jnp.int32),
        pltpu.VMEM((CH,), jnp.float32),
    ],
)
def _scw2(src_hbm, dst_hbm, a2t_hbm, w2_hbm, av, dv, src_v, dst_v, wv):
    wid = lax.axis_index("s") * NC + lax.axis_index("c")
    pltpu.sync_copy(a2t_hbm.at[0], av)
    pltpu.sync_copy(a2t_hbm.at[1], dv)
    ebase = wid * (E_PAD // NT)

    def chunk(k, _):
        e0 = ebase + k * CH
        pltpu.sync_copy(src_hbm.at[pl.ds(e0, CH)], src_v)
        pltpu.sync_copy(dst_hbm.at[pl.ds(e0, CH)], dst_v)

        def grp(g, _):
            s16 = src_v[pl.ds(g * L, L)]
            d16 = dst_v[pl.ds(g * L, L)]
            xv = plsc.load_gather(av, [s16]) + plsc.load_gather(dv, [d16])
            wv[pl.ds(g * L, L)] = jnp.exp(jnp.maximum(xv, 0.2 * xv))
            return 0

        lax.fori_loop(0, CH // L, grp, 0)
        pltpu.sync_copy(wv, w2_hbm.at[pl.ds(e0, CH)])
        return 0

    lax.fori_loop(0, E_PAD // NT // CH, chunk, 0)


def kernel(x, edge_index, W1, att_src1, att_dst1, b1,
           W2, att_src2, att_dst2, b2):
    ei = edge_index.astype(jnp.int32)
    loops = jnp.arange(N, dtype=jnp.int32)
    npad = E_PAD - E
    src = jnp.concatenate([ei[0], loops, jnp.zeros((npad,), jnp.int32)])
    dst = jnp.concatenate([ei[1], loops, jnp.full((npad,), N, jnp.int32)])

    eye8 = jnp.eye(8, dtype=jnp.float32)
    S1 = (att_src1[:, :, None] * eye8[:, None, :]).reshape(64, 8)
    D1 = (att_dst1[:, :, None] * eye8[:, None, :]).reshape(64, 8)
    REP = jnp.kron(eye8, jnp.ones((1, 8), jnp.float32))

    hT, asrcT, adstT = _tc1(x, W1, S1, D1)
    wT = _scw1(src, dst, asrcT, adstT)
    acc1 = _scm1(src, dst, hT, wT)
    sp1 = _scs1(dst, wT)
    h2T, a2T = _tc2(acc1, sp1, b1.reshape(1, 64), REP,
                    W2, att_src2.reshape(7, 1), att_dst2.reshape(7, 1))
    w2 = _scw2(src, dst, a2T)
    acc2 = _scm2(src, dst, h2T, w2)
    return _tc3(acc2, b2.reshape(1, 7))[0]


# speedup vs baseline: 41.1919x; 1.7672x over previous
import functools

import jax
import jax.numpy as jnp
from jax import lax
from jax.experimental import pallas as pl
from jax.experimental.pallas import tpu as pltpu
from jax.experimental.pallas import tpu_sc as plsc

N = 50000
IN_DIM = 1433
NC, NS, L = 2, 16, 16
NT = NC * NS
RB = 512
N_PAD = 50176
E = N + 800000
E_PAD = 851968
EQ = E_PAD // 4
EH = E_PAD // 2
CH = 2048
CHM = 4096
_CP = pltpu.CompilerParams(needs_layout_passes=False)
_MESH = plsc.VectorSubcoreMesh(core_axis_name="c", subcore_axis_name="s",
                               num_cores=NC, num_subcores=NS)


def _tc1_body(x_ref, w1_ref, s1_ref, d1_ref, ht_ref, ast_ref, adt_ref):
    h = jnp.dot(x_ref[...], w1_ref[...], preferred_element_type=jnp.float32)
    asrc = jnp.dot(h, s1_ref[...], preferred_element_type=jnp.float32)
    adst = jnp.dot(h, d1_ref[...], preferred_element_type=jnp.float32)
    ht_ref[...] = jnp.transpose(h)
    ast_ref[...] = jnp.transpose(asrc)
    adt_ref[...] = jnp.transpose(adst)


_tc1 = pl.pallas_call(
    _tc1_body,
    grid=(N_PAD // RB,),
    in_specs=[
        pl.BlockSpec((RB, IN_DIM), lambda i: (i, 0)),
        pl.BlockSpec((IN_DIM, 64), lambda i: (0, 0)),
        pl.BlockSpec((64, 8), lambda i: (0, 0)),
        pl.BlockSpec((64, 8), lambda i: (0, 0)),
    ],
    out_specs=[
        pl.BlockSpec((64, RB), lambda i: (0, i)),
        pl.BlockSpec((8, RB), lambda i: (0, i)),
        pl.BlockSpec((8, RB), lambda i: (0, i)),
    ],
    out_shape=[
        jax.ShapeDtypeStruct((64, N_PAD), jnp.float32),
        jax.ShapeDtypeStruct((8, N_PAD), jnp.float32),
        jax.ShapeDtypeStruct((8, N_PAD), jnp.float32),
    ],
)


def _tc2_body(acc_ref, sp_ref, b1_ref, rep_ref, w2_ref, s2_ref, d2_ref,
              h2t_ref, a2t_ref):
    a = acc_ref[...]
    num = jnp.transpose(a[0] + a[1])
    sp = sp_ref[...]
    sden = jnp.transpose(sp[0] + sp[1] + sp[2] + sp[3])
    srep = jnp.dot(sden, rep_ref[...], preferred_element_type=jnp.float32)
    out1 = num / (srep + 1e-16) + b1_ref[...]
    hh = jnp.where(out1 > 0, out1, jnp.exp(out1) - 1.0)
    h2 = jnp.dot(hh, w2_ref[...], preferred_element_type=jnp.float32)
    asrc2 = jnp.dot(h2, s2_ref[...], preferred_element_type=jnp.float32)
    adst2 = jnp.dot(h2, d2_ref[...], preferred_element_type=jnp.float32)
    h2t_ref[...] = jnp.transpose(h2)
    a2t_ref[...] = jnp.transpose(jnp.concatenate([asrc2, adst2], axis=1))


_tc2 = pl.pallas_call(
    _tc2_body,
    grid=(N_PAD // RB,),
    in_specs=[
        pl.BlockSpec((2, 64, RB), lambda i: (0, 0, i)),
        pl.BlockSpec((4, 8, RB), lambda i: (0, 0, i)),
        pl.BlockSpec((1, 64), lambda i: (0, 0)),
        pl.BlockSpec((8, 64), lambda i: (0, 0)),
        pl.BlockSpec((64, 7), lambda i: (0, 0)),
        pl.BlockSpec((7, 1), lambda i: (0, 0)),
        pl.BlockSpec((7, 1), lambda i: (0, 0)),
    ],
    out_specs=[
        pl.BlockSpec((7, RB), lambda i: (0, i)),
        pl.BlockSpec((2, RB), lambda i: (0, i)),
    ],
    out_shape=[
        jax.ShapeDtypeStruct((7, N_PAD), jnp.float32),
        jax.ShapeDtypeStruct((2, N_PAD), jnp.float32),
    ],
)


def _tc3_body(acc_ref, b2_ref, out_ref):
    a = acc_ref[...]
    t = a[0] + a[1] + a[2] + a[3]
    o = jnp.transpose(t[:7] / (t[7:8] + 1e-16)) + b2_ref[...]
    m = jnp.max(o, axis=1, keepdims=True)
    z = o - m
    lse = jnp.log(jnp.sum(jnp.exp(z), axis=1, keepdims=True))
    out_ref[...] = z - lse


_tc3 = pl.pallas_call(
    _tc3_body,
    grid=(N_PAD // RB,),
    in_specs=[
        pl.BlockSpec((4, 8, RB), lambda i: (0, 0, i)),
        pl.BlockSpec((1, 7), lambda i: (0, 0)),
    ],
    out_specs=[pl.BlockSpec((RB, 7), lambda i: (i, 0))],
    out_shape=[jax.ShapeDtypeStruct((N, 7), jnp.float32)],
)


def _zero_1d(ref, nwords):
    z16 = jnp.zeros((L,), jnp.float32)

    def zb(r, _):
        ref[pl.ds(r * L, L)] = z16
        return 0

    lax.fori_loop(0, nwords // L, zb, 0)


@functools.partial(
    pl.kernel,
    out_type=jax.ShapeDtypeStruct((8, E_PAD), jnp.float32),
    mesh=_MESH,
    compiler_params=_CP,
    scratch_types=[
        pltpu.VMEM((N_PAD,), jnp.float32),
        pltpu.VMEM((N_PAD,), jnp.float32),
        pltpu.VMEM((CH,), jnp.int32),
        pltpu.VMEM((CH,), jnp.int32),
        pltpu.VMEM((CH,), jnp.float32),
    ],
)
def _scw1(src_hbm, dst_hbm, ast_hbm, adt_hbm, wt_hbm,
          av, dv, src_v, dst_v, wv):
    wid = lax.axis_index("s") * NC + lax.axis_index("c")
    task_h = wid // 4
    task_q = wid - task_h * 4
    pltpu.sync_copy(ast_hbm.at[task_h], av)
    pltpu.sync_copy(adt_hbm.at[task_h], dv)
    ebase = task_q * EQ

    def chunk(k, _):
        e0 = ebase + k * CH
        pltpu.sync_copy(src_hbm.at[pl.ds(e0, CH)], src_v)
        pltpu.sync_copy(dst_hbm.at[pl.ds(e0, CH)], dst_v)

        def grp(g, _):
            s16 = src_v[pl.ds(g * L, L)]
            d16 = dst_v[pl.ds(g * L, L)]
            xv = plsc.load_gather(av, [s16]) + plsc.load_gather(dv, [d16])
            wv[pl.ds(g * L, L)] = jnp.exp(jnp.maximum(xv, 0.2 * xv))
            return 0

        lax.fori_loop(0, CH // L, grp, 0)
        pltpu.sync_copy(wv, wt_hbm.at[task_h, pl.ds(e0, CH)])
        return 0

    lax.fori_loop(0, EQ // CH, chunk, 0)


@functools.partial(
    pl.kernel,
    out_type=jax.ShapeDtypeStruct((4, 8, N_PAD), jnp.float32),
    mesh=_MESH,
    compiler_params=_CP,
    scratch_types=[
        pltpu.VMEM((N_PAD,), jnp.float32),
        pltpu.VMEM((CH,), jnp.int32),
        pltpu.VMEM((CH,), jnp.float32),
    ],
)
def _scs1(dst_hbm, wt_hbm, out_hbm, accv, dst_v, wv):
    wid = lax.axis_index("s") * NC + lax.axis_index("c")
    task_h = wid // 4
    task_q = wid - task_h * 4
    _zero_1d(accv, N_PAD)
    ebase = task_q * EQ

    def chunk(k, _):
        e0 = ebase + k * CH
        pltpu.sync_copy(dst_hbm.at[pl.ds(e0, CH)], dst_v)
        pltpu.sync_copy(wt_hbm.at[task_h, pl.ds(e0, CH)], wv)

        def grp(g, _):
            d16 = dst_v[pl.ds(g * L, L)]
            w = wv[pl.ds(g * L, L)]
            plsc.addupdate_scatter(accv, [d16], w)
            return 0

        lax.fori_loop(0, CH // L, grp, 0)
        return 0

    lax.fori_loop(0, EQ // CH, chunk, 0)
    pltpu.sync_copy(accv, out_hbm.at[task_q, task_h])


@functools.partial(
    pl.kernel,
    out_type=jax.ShapeDtypeStruct((2, 64, N_PAD), jnp.float32),
    mesh=_MESH,
    compiler_params=_CP,
    scratch_types=[
        pltpu.VMEM((N_PAD,), jnp.float32),
        pltpu.VMEM((N_PAD,), jnp.float32),
        pltpu.VMEM((CHM,), jnp.int32),
        pltpu.VMEM((CHM,), jnp.int32),
        pltpu.VMEM((CHM,), jnp.float32),
    ],
)
def _scm1(src_hbm, dst_hbm, ht_hbm, wt_hbm, out_hbm,
          tabv, accv, src_v, dst_v, wv):
    wid = lax.axis_index("s") * NC + lax.axis_index("c")

    for t in range(4):
        tid = wid * 4 + t
        ch = tid // 2
        half = tid - ch * 2
        head = ch // 8
        _zero_1d(accv, N_PAD)
        pltpu.sync_copy(ht_hbm.at[ch], tabv)
        ebase = half * EH

        def chunk(k, _):
            e0 = ebase + k * CHM
            pltpu.sync_copy(src_hbm.at[pl.ds(e0, CHM)], src_v)
            pltpu.sync_copy(dst_hbm.at[pl.ds(e0, CHM)], dst_v)
            pltpu.sync_copy(wt_hbm.at[head, pl.ds(e0, CHM)], wv)

            def grp(g, _):
                s16 = src_v[pl.ds(g * L, L)]
                d16 = dst_v[pl.ds(g * L, L)]
                w = wv[pl.ds(g * L, L)]
                v = plsc.load_gather(tabv, [s16])
                plsc.addupdate_scatter(accv, [d16], v * w)
                return 0

            lax.fori_loop(0, CHM // L, grp, 0)
            return 0

        lax.fori_loop(0, EH // CHM, chunk, 0)
        pltpu.sync_copy(accv, out_hbm.at[half, ch])


@functools.partial(
    pl.kernel,
    out_type=jax.ShapeDtypeStruct((4, 8, N_PAD), jnp.float32),
    mesh=_MESH,
    compiler_params=_CP,
    scratch_types=[
        pltpu.VMEM((N_PAD,), jnp.float32),
        pltpu.VMEM((N_PAD,), jnp.float32),
        pltpu.VMEM((CH,), jnp.int32),
        pltpu.VMEM((CH,), jnp.int32),
        pltpu.VMEM((CH,), jnp.float32),
    ],
)
def _scm2(src_hbm, dst_hbm, h2t_hbm, w2_hbm, out_hbm,
          tabv, accv, src_v, dst_v, wv):
    wid = lax.axis_index("s") * NC + lax.axis_index("c")
    task_c = wid // 4
    task_q = wid - task_c * 4
    _zero_1d(accv, N_PAD)

    @pl.when(task_c < 7)
    def _():
        pltpu.sync_copy(h2t_hbm.at[task_c], tabv)

    ebase = task_q * EQ

    def chunk(k, _):
        e0 = ebase + k * CH
        pltpu.sync_copy(src_hbm.at[pl.ds(e0, CH)], src_v)
        pltpu.sync_copy(dst_hbm.at[pl.ds(e0, CH)], dst_v)
        pltpu.sync_copy(w2_hbm.at[pl.ds(e0, CH)], wv)

        @pl.when(task_c < 7)
        def _():
            def grp(g, _):
                s16 = src_v[pl.ds(g * L, L)]
                d16 = dst_v[pl.ds(g * L, L)]
                w = wv[pl.ds(g * L, L)]
                v = plsc.load_gather(tabv, [s16])
                plsc.addupdate_scatter(accv, [d16], v * w)
                return 0

            lax.fori_loop(0, CH // L, grp, 0)

        @pl.when(task_c == 7)
        def _():
            def grp(g, _):
                d16 = dst_v[pl.ds(g * L, L)]
                w = wv[pl.ds(g * L, L)]
                plsc.addupdate_scatter(accv, [d16], w)
                return 0

            lax.fori_loop(0, CH // L, grp, 0)

        return 0

    lax.fori_loop(0, EQ // CH, chunk, 0)
    pltpu.sync_copy(accv, out_hbm.at[task_q, task_c])


@functools.partial(
    pl.kernel,
    out_type=jax.ShapeDtypeStruct((E_PAD,), jnp.float32),
    mesh=_MESH,
    compiler_params=_CP,
    scratch_types=[
        pltpu.VMEM((N_PAD,), jnp.float32),
        pltpu.VMEM((N_PAD,), jnp.float32),
        pltpu.VMEM((CH,), jnp.int32),
        pltpu.VMEM((CH,), jnp.int32),
        pltpu.VMEM((CH,), jnp.float32),
    ],
)
def _scw2(src_hbm, dst_hbm, a2t_hbm, w2_hbm, av, dv, src_v, dst_v, wv):
    wid = lax.axis_index("s") * NC + lax.axis_index("c")
    pltpu.sync_copy(a2t_hbm.at[0], av)
    pltpu.sync_copy(a2t_hbm.at[1], dv)
    ebase = wid * (E_PAD // NT)

    def chunk(k, _):
        e0 = ebase + k * CH
        pltpu.sync_copy(src_hbm.at[pl.ds(e0, CH)], src_v)
        pltpu.sync_copy(dst_hbm.at[pl.ds(e0, CH)], dst_v)

        def grp(g, _):
            s16 = src_v[pl.ds(g * L, L)]
            d16 = dst_v[pl.ds(g * L, L)]
            xv = plsc.load_gather(av, [s16]) + plsc.load_gather(dv, [d16])
            wv[pl.ds(g * L, L)] = jnp.exp(jnp.maximum(xv, 0.2 * xv))
            return 0

        lax.fori_loop(0, CH // L, grp, 0)
        pltpu.sync_copy(wv, w2_hbm.at[pl.ds(e0, CH)])
        return 0

    lax.fori_loop(0, E_PAD // NT // CH, chunk, 0)


def kernel(x, edge_index, W1, att_src1, att_dst1, b1,
           W2, att_src2, att_dst2, b2):
    ei = edge_index.astype(jnp.int32)
    loops = jnp.arange(N, dtype=jnp.int32)
    npad = E_PAD - E
    src = jnp.concatenate([ei[0], loops, jnp.zeros((npad,), jnp.int32)])
    dst = jnp.concatenate([ei[1], loops, jnp.full((npad,), N, jnp.int32)])

    eye8 = jnp.eye(8, dtype=jnp.float32)
    S1 = (att_src1[:, :, None] * eye8[:, None, :]).reshape(64, 8)
    D1 = (att_dst1[:, :, None] * eye8[:, None, :]).reshape(64, 8)
    REP = jnp.kron(eye8, jnp.ones((1, 8), jnp.float32))

    hT, asrcT, adstT = _tc1(x, W1, S1, D1)
    wT = _scw1(src, dst, asrcT, adstT)
    acc1 = _scm1(src, dst, hT, wT)
    sp1 = _scs1(dst, wT)
    h2T, a2T = _tc2(acc1, sp1, b1.reshape(1, 64), REP,
                    W2, att_src2.reshape(7, 1), att_dst2.reshape(7, 1))
    w2 = _scw2(src, dst, a2T)
    acc2 = _scm2(src, dst, h2T, w2)
    return _tc3(acc2, b2.reshape(1, 7))[0]


# speedup vs baseline: 49.9808x; 1.2134x over previous
import functools

import jax
import jax.numpy as jnp
from jax import lax
from jax.experimental import pallas as pl
from jax.experimental.pallas import tpu as pltpu
from jax.experimental.pallas import tpu_sc as plsc

N = 50000
IN_DIM = 1433
NC, NS, L = 2, 16, 16
NT = NC * NS
RB = 512
N_PAD = 50176
E = N + 800000
E_PAD = 851968
EQ = E_PAD // 4
EH = E_PAD // 2
CH = 2048
CHM = 4096
_CP = pltpu.CompilerParams(needs_layout_passes=False)
_MESH = plsc.VectorSubcoreMesh(core_axis_name="c", subcore_axis_name="s",
                               num_cores=NC, num_subcores=NS)


def _tc1_body(x_ref, w1_ref, s1_ref, d1_ref, ht_ref, ast_ref, adt_ref):
    h = jnp.dot(x_ref[...], w1_ref[...], preferred_element_type=jnp.float32)
    asrc = jnp.dot(h, s1_ref[...], preferred_element_type=jnp.float32)
    adst = jnp.dot(h, d1_ref[...], preferred_element_type=jnp.float32)
    ht_ref[...] = jnp.transpose(h)
    ast_ref[...] = jnp.transpose(asrc)
    adt_ref[...] = jnp.transpose(adst)


_tc1 = pl.pallas_call(
    _tc1_body,
    grid=(N_PAD // RB,),
    in_specs=[
        pl.BlockSpec((RB, IN_DIM), lambda i: (i, 0)),
        pl.BlockSpec((IN_DIM, 64), lambda i: (0, 0)),
        pl.BlockSpec((64, 8), lambda i: (0, 0)),
        pl.BlockSpec((64, 8), lambda i: (0, 0)),
    ],
    out_specs=[
        pl.BlockSpec((64, RB), lambda i: (0, i)),
        pl.BlockSpec((8, RB), lambda i: (0, i)),
        pl.BlockSpec((8, RB), lambda i: (0, i)),
    ],
    out_shape=[
        jax.ShapeDtypeStruct((64, N_PAD), jnp.float32),
        jax.ShapeDtypeStruct((8, N_PAD), jnp.float32),
        jax.ShapeDtypeStruct((8, N_PAD), jnp.float32),
    ],
)


def _tc2_body(acc_ref, sp_ref, b1_ref, rep_ref, w2_ref, s2_ref, d2_ref,
              h2t_ref, a2t_ref):
    a = acc_ref[...]
    num = jnp.transpose(a[0] + a[1])
    sp = sp_ref[...]
    sden = jnp.transpose(sp[0] + sp[1] + sp[2] + sp[3])
    srep = jnp.dot(sden, rep_ref[...], preferred_element_type=jnp.float32)
    out1 = num / (srep + 1e-16) + b1_ref[...]
    hh = jnp.where(out1 > 0, out1, jnp.exp(out1) - 1.0)
    h2 = jnp.dot(hh, w2_ref[...], preferred_element_type=jnp.float32)
    asrc2 = jnp.dot(h2, s2_ref[...], preferred_element_type=jnp.float32)
    adst2 = jnp.dot(h2, d2_ref[...], preferred_element_type=jnp.float32)
    h2t_ref[...] = jnp.transpose(h2)
    a2t_ref[...] = jnp.transpose(jnp.concatenate([asrc2, adst2], axis=1))


_tc2 = pl.pallas_call(
    _tc2_body,
    grid=(N_PAD // RB,),
    in_specs=[
        pl.BlockSpec((2, 64, RB), lambda i: (0, 0, i)),
        pl.BlockSpec((4, 8, RB), lambda i: (0, 0, i)),
        pl.BlockSpec((1, 64), lambda i: (0, 0)),
        pl.BlockSpec((8, 64), lambda i: (0, 0)),
        pl.BlockSpec((64, 7), lambda i: (0, 0)),
        pl.BlockSpec((7, 1), lambda i: (0, 0)),
        pl.BlockSpec((7, 1), lambda i: (0, 0)),
    ],
    out_specs=[
        pl.BlockSpec((7, RB), lambda i: (0, i)),
        pl.BlockSpec((2, RB), lambda i: (0, i)),
    ],
    out_shape=[
        jax.ShapeDtypeStruct((7, N_PAD), jnp.float32),
        jax.ShapeDtypeStruct((2, N_PAD), jnp.float32),
    ],
)


def _tc3_body(acc_ref, b2_ref, out_ref):
    a = acc_ref[...]
    t = a[0] + a[1] + a[2] + a[3]
    o = jnp.transpose(t[:7] / (t[7:8] + 1e-16)) + b2_ref[...]
    m = jnp.max(o, axis=1, keepdims=True)
    z = o - m
    lse = jnp.log(jnp.sum(jnp.exp(z), axis=1, keepdims=True))
    out_ref[...] = z - lse


_tc3 = pl.pallas_call(
    _tc3_body,
    grid=(N_PAD // RB,),
    in_specs=[
        pl.BlockSpec((4, 8, RB), lambda i: (0, 0, i)),
        pl.BlockSpec((1, 7), lambda i: (0, 0)),
    ],
    out_specs=[pl.BlockSpec((RB, 7), lambda i: (i, 0))],
    out_shape=[jax.ShapeDtypeStruct((N, 7), jnp.float32)],
)


def _zero_1d(ref, nwords):
    z16 = jnp.zeros((L,), jnp.float32)

    def zb(r, _):
        ref[pl.ds(r * L, L)] = z16
        return 0

    lax.fori_loop(0, nwords // L, zb, 0)


@functools.partial(
    pl.kernel,
    out_type=jax.ShapeDtypeStruct((8, E_PAD), jnp.float32),
    mesh=_MESH,
    compiler_params=_CP,
    scratch_types=[
        pltpu.VMEM((N_PAD,), jnp.float32),
        pltpu.VMEM((N_PAD,), jnp.float32),
        pltpu.VMEM((CH,), jnp.int32),
        pltpu.VMEM((CH,), jnp.int32),
        pltpu.VMEM((CH,), jnp.float32),
    ],
)
def _scw1(src_hbm, dst_hbm, ast_hbm, adt_hbm, wt_hbm,
          av, dv, src_v, dst_v, wv):
    wid = lax.axis_index("s") * NC + lax.axis_index("c")
    task_h = wid // 4
    task_q = wid - task_h * 4
    pltpu.sync_copy(ast_hbm.at[task_h], av)
    pltpu.sync_copy(adt_hbm.at[task_h], dv)
    ebase = task_q * EQ

    def chunk(k, _):
        e0 = ebase + k * CH
        pltpu.sync_copy(src_hbm.at[pl.ds(e0, CH)], src_v)
        pltpu.sync_copy(dst_hbm.at[pl.ds(e0, CH)], dst_v)

        def grp(g, _):
            s16 = src_v[pl.ds(g * L, L)]
            d16 = dst_v[pl.ds(g * L, L)]
            xv = plsc.load_gather(av, [s16]) + plsc.load_gather(dv, [d16])
            wv[pl.ds(g * L, L)] = jnp.exp(jnp.maximum(xv, 0.2 * xv))
            return 0

        lax.fori_loop(0, CH // L, grp, 0)
        pltpu.sync_copy(wv, wt_hbm.at[task_h, pl.ds(e0, CH)])
        return 0

    lax.fori_loop(0, EQ // CH, chunk, 0)


@functools.partial(
    pl.kernel,
    out_type=jax.ShapeDtypeStruct((4, 8, N_PAD), jnp.float32),
    mesh=_MESH,
    compiler_params=_CP,
    scratch_types=[
        pltpu.VMEM((N_PAD,), jnp.float32),
        pltpu.VMEM((CH,), jnp.int32),
        pltpu.VMEM((CH,), jnp.float32),
    ],
)
def _scs1(dst_hbm, wt_hbm, out_hbm, accv, dst_v, wv):
    wid = lax.axis_index("s") * NC + lax.axis_index("c")
    task_h = wid // 4
    task_q = wid - task_h * 4
    _zero_1d(accv, N_PAD)
    ebase = task_q * EQ

    def chunk(k, _):
        e0 = ebase + k * CH
        pltpu.sync_copy(dst_hbm.at[pl.ds(e0, CH)], dst_v)
        pltpu.sync_copy(wt_hbm.at[task_h, pl.ds(e0, CH)], wv)

        def grp(g, _):
            d16 = dst_v[pl.ds(g * L, L)]
            w = wv[pl.ds(g * L, L)]
            plsc.addupdate_scatter(accv, [d16], w)
            return 0

        lax.fori_loop(0, CH // L, grp, 0)
        return 0

    lax.fori_loop(0, EQ // CH, chunk, 0)
    pltpu.sync_copy(accv, out_hbm.at[task_q, task_h])


@functools.partial(
    pl.kernel,
    out_type=jax.ShapeDtypeStruct((2, 64, N_PAD), jnp.float32),
    mesh=_MESH,
    compiler_params=_CP,
    scratch_types=[
        pltpu.VMEM((N_PAD,), jnp.float32),
        pltpu.VMEM((N_PAD,), jnp.float32),
        pltpu.VMEM((2, CHM), jnp.int32),
        pltpu.VMEM((2, CHM), jnp.int32),
        pltpu.VMEM((2, CHM), jnp.float32),
        pltpu.SemaphoreType.DMA,
        pltpu.SemaphoreType.DMA,
    ],
)
def _scm1(src_hbm, dst_hbm, ht_hbm, wt_hbm, out_hbm,
          tabv, accv, src_v, dst_v, wv, sem0, sem1):
    wid = lax.axis_index("s") * NC + lax.axis_index("c")
    sems = (sem0, sem1)
    nch = EH // CHM

    for t in range(4):
        tid = wid * 4 + t
        ch = tid // 2
        half = tid - ch * 2
        head = ch // 8
        _zero_1d(accv, N_PAD)
        pltpu.sync_copy(ht_hbm.at[ch], tabv)
        ebase = half * EH

        def issue(e0, b):
            pltpu.async_copy(src_hbm.at[pl.ds(e0, CHM)], src_v.at[b], sems[b])
            pltpu.async_copy(dst_hbm.at[pl.ds(e0, CHM)], dst_v.at[b], sems[b])
            pltpu.async_copy(wt_hbm.at[head, pl.ds(e0, CHM)], wv.at[b],
                             sems[b])

        def drain(b):
            pltpu.make_async_copy(src_hbm.at[pl.ds(0, CHM)], src_v.at[b],
                                  sems[b]).wait()
            pltpu.make_async_copy(dst_hbm.at[pl.ds(0, CHM)], dst_v.at[b],
                                  sems[b]).wait()
            pltpu.make_async_copy(wt_hbm.at[head, pl.ds(0, CHM)], wv.at[b],
                                  sems[b]).wait()

        def compute(b):
            def grp(g, _):
                s16 = src_v[b, pl.ds(g * L, L)]
                d16 = dst_v[b, pl.ds(g * L, L)]
                w = wv[b, pl.ds(g * L, L)]
                v = plsc.load_gather(tabv, [s16])
                plsc.addupdate_scatter(accv, [d16], v * w)
                return 0

            lax.fori_loop(0, CHM // L, grp, 0)

        issue(ebase, 0)

        def pair(kk, _):
            issue(ebase + (2 * kk + 1) * CHM, 1)
            drain(0)
            compute(0)
            nxt = jnp.where(2 * kk + 2 < nch, (2 * kk + 2) * CHM, 0)
            issue(ebase + nxt, 0)
            drain(1)
            compute(1)
            return 0

        lax.fori_loop(0, nch // 2, pair, 0)
        drain(0)
        pltpu.sync_copy(accv, out_hbm.at[half, ch])


@functools.partial(
    pl.kernel,
    out_type=jax.ShapeDtypeStruct((4, 8, N_PAD), jnp.float32),
    mesh=_MESH,
    compiler_params=_CP,
    scratch_types=[
        pltpu.VMEM((N_PAD,), jnp.float32),
        pltpu.VMEM((N_PAD,), jnp.float32),
        pltpu.VMEM((CH,), jnp.int32),
        pltpu.VMEM((CH,), jnp.int32),
        pltpu.VMEM((CH,), jnp.float32),
    ],
)
def _scm2(src_hbm, dst_hbm, h2t_hbm, w2_hbm, out_hbm,
          tabv, accv, src_v, dst_v, wv):
    wid = lax.axis_index("s") * NC + lax.axis_index("c")
    task_c = wid // 4
    task_q = wid - task_c * 4
    _zero_1d(accv, N_PAD)

    @pl.when(task_c < 7)
    def _():
        pltpu.sync_copy(h2t_hbm.at[task_c], tabv)

    ebase = task_q * EQ

    def chunk(k, _):
        e0 = ebase + k * CH
        pltpu.sync_copy(src_hbm.at[pl.ds(e0, CH)], src_v)
        pltpu.sync_copy(dst_hbm.at[pl.ds(e0, CH)], dst_v)
        pltpu.sync_copy(w2_hbm.at[pl.ds(e0, CH)], wv)

        @pl.when(task_c < 7)
        def _():
            def grp(g, _):
                s16 = src_v[pl.ds(g * L, L)]
                d16 = dst_v[pl.ds(g * L, L)]
                w = wv[pl.ds(g * L, L)]
                v = plsc.load_gather(tabv, [s16])
                plsc.addupdate_scatter(accv, [d16], v * w)
                return 0

            lax.fori_loop(0, CH // L, grp, 0)

        @pl.when(task_c == 7)
        def _():
            def grp(g, _):
                d16 = dst_v[pl.ds(g * L, L)]
                w = wv[pl.ds(g * L, L)]
                plsc.addupdate_scatter(accv, [d16], w)
                return 0

            lax.fori_loop(0, CH // L, grp, 0)

        return 0

    lax.fori_loop(0, EQ // CH, chunk, 0)
    pltpu.sync_copy(accv, out_hbm.at[task_q, task_c])


@functools.partial(
    pl.kernel,
    out_type=jax.ShapeDtypeStruct((E_PAD,), jnp.float32),
    mesh=_MESH,
    compiler_params=_CP,
    scratch_types=[
        pltpu.VMEM((N_PAD,), jnp.float32),
        pltpu.VMEM((N_PAD,), jnp.float32),
        pltpu.VMEM((CH,), jnp.int32),
        pltpu.VMEM((CH,), jnp.int32),
        pltpu.VMEM((CH,), jnp.float32),
    ],
)
def _scw2(src_hbm, dst_hbm, a2t_hbm, w2_hbm, av, dv, src_v, dst_v, wv):
    wid = lax.axis_index("s") * NC + lax.axis_index("c")
    pltpu.sync_copy(a2t_hbm.at[0], av)
    pltpu.sync_copy(a2t_hbm.at[1], dv)
    ebase = wid * (E_PAD // NT)

    def chunk(k, _):
        e0 = ebase + k * CH
        pltpu.sync_copy(src_hbm.at[pl.ds(e0, CH)], src_v)
        pltpu.sync_copy(dst_hbm.at[pl.ds(e0, CH)], dst_v)

        def grp(g, _):
            s16 = src_v[pl.ds(g * L, L)]
            d16 = dst_v[pl.ds(g * L, L)]
            xv = plsc.load_gather(av, [s16]) + plsc.load_gather(dv, [d16])
            wv[pl.ds(g * L, L)] = jnp.exp(jnp.maximum(xv, 0.2 * xv))
            return 0

        lax.fori_loop(0, CH // L, grp, 0)
        pltpu.sync_copy(wv, w2_hbm.at[pl.ds(e0, CH)])
        return 0

    lax.fori_loop(0, E_PAD // NT // CH, chunk, 0)


def kernel(x, edge_index, W1, att_src1, att_dst1, b1,
           W2, att_src2, att_dst2, b2):
    ei = edge_index.astype(jnp.int32)
    loops = jnp.arange(N, dtype=jnp.int32)
    npad = E_PAD - E
    src = jnp.concatenate([ei[0], loops, jnp.zeros((npad,), jnp.int32)])
    dst = jnp.concatenate([ei[1], loops, jnp.full((npad,), N, jnp.int32)])

    eye8 = jnp.eye(8, dtype=jnp.float32)
    S1 = (att_src1[:, :, None] * eye8[:, None, :]).reshape(64, 8)
    D1 = (att_dst1[:, :, None] * eye8[:, None, :]).reshape(64, 8)
    REP = jnp.kron(eye8, jnp.ones((1, 8), jnp.float32))

    hT, asrcT, adstT = _tc1(x, W1, S1, D1)
    wT = _scw1(src, dst, asrcT, adstT)
    acc1 = _scm1(src, dst, hT, wT)
    sp1 = _scs1(dst, wT)
    h2T, a2T = _tc2(acc1, sp1, b1.reshape(1, 64), REP,
                    W2, att_src2.reshape(7, 1), att_dst2.reshape(7, 1))
    w2 = _scw2(src, dst, a2T)
    acc2 = _scm2(src, dst, h2T, w2)
    return _tc3(acc2, b2.reshape(1, 7))[0]
